# Initial kernel scaffold; baseline (speedup 1.0000x reference)
import jax
import jax.numpy as jnp
from jax.experimental import pallas as pl


def kernel(b, xref, edge_index, edge_weights):
    raise NotImplementedError("write your pallas kernel here")


# speedup vs baseline: 5.7442x; 5.7442x over previous
import functools

import jax
import jax.numpy as jnp
from jax import lax
from jax.experimental import pallas as pl
from jax.experimental.pallas import tpu as pltpu
from jax.experimental.pallas import tpu_sc as plsc

N_NODES = 10000
D_FEAT = 128
N_EDGES = 320000
CGLS_IT = 10
EPS = 0.01

NC = 2
NS = 16
L = 16
NW = NC * NS
CPT = D_FEAT // NW
CHUNK = 6400
NCHUNKS = N_EDGES // CHUNK
GROUPS = CHUNK // L
NODE_VECS = N_NODES // L
UNROLL = 4


def _take16(x, idx):
    dnums = lax.GatherDimensionNumbers(
        offset_dims=(), collapsed_slice_dims=(0,), start_index_map=(0,))
    return lax.gather(x, idx[:, None], dnums, (1,),
                      mode=lax.GatherScatterMode.PROMISE_IN_BOUNDS)


def _spmv_body(x_hbm, pidx_hbm, w_hbm, flags_hbm, out_hbm,
               psum_hbm,
               x_v, out_v, p0_v, p1_v, w0_v, w1_v, f_v, acc_v,
               sem0, sem1):
    wid = lax.axis_index("s") * NC + lax.axis_index("c")
    base_flat = wid * (CPT * N_NODES)
    pv = (p0_v, p1_v)
    wv_ = (w0_v, w1_v)
    sems = (sem0, sem1)

    def start_chunk(ci, b):
        base = ci * CHUNK
        pltpu.async_copy(pidx_hbm.at[pl.ds(base, CHUNK)], pv[b], sems[b])
        pltpu.async_copy(w_hbm.at[pl.ds(base, CHUNK)], wv_[b], sems[b])

    def wait_chunk(ci, b):
        base = ci * CHUNK
        pltpu.make_async_copy(pidx_hbm.at[pl.ds(base, CHUNK)], pv[b],
                              sems[b]).wait()
        pltpu.make_async_copy(w_hbm.at[pl.ds(base, CHUNK)], wv_[b],
                              sems[b]).wait()

    start_chunk(jnp.int32(0), 0)

    pltpu.sync_copy(x_hbm.at[pl.ds(base_flat, CPT * N_NODES)], x_v)
    pltpu.sync_copy(flags_hbm, f_v)

    @plsc.parallel_loop(jnp.int32(0), jnp.int32(CPT * NODE_VECS),
                        jnp.int32(1), unroll=8)
    def _zero(j):
        out_v[pl.ds(j * L, L)] = jnp.zeros((L,), jnp.float32)

    iota = lax.iota(jnp.int32, L)
    lane_lt = iota < (L - 1)
    rot_prev = (iota + (L - 1)) % L
    rot_next = (iota + 1) % L
    xslice = [x_v.at[pl.ds(c * N_NODES, N_NODES)] for c in range(CPT)]
    oslice = [out_v.at[pl.ds(c * N_NODES, N_NODES)] for c in range(CPT)]

    def process(ci, p_e, w_e):
        fv = f_v[pl.ds(ci * L, L)]
        dup = jnp.max(fv) > 0

        @pl.when(jnp.logical_not(dup))
        def _():
            @plsc.parallel_loop(jnp.int32(0), jnp.int32(GROUPS),
                                jnp.int32(1), unroll=8)
            def _blk(j):
                o = j * L
                pk = p_e[pl.ds(o, L)]
                gi = pk & jnp.int32(0x3FFF)
                si = lax.shift_right_logical(pk, jnp.int32(14))
                wvv = w_e[pl.ds(o, L)]
                for c in range(CPT):
                    v = plsc.load_gather(xslice[c], [gi]) * wvv
                    plsc.addupdate_scatter(oslice[c], [si], v)

        @pl.when(dup)
        def _():
            def blk(j, _):
                o = j * L
                pk = p_e[pl.ds(o, L)]
                gi = pk & jnp.int32(0x3FFF)
                si = lax.shift_right_logical(pk, jnp.int32(14))
                wvv = w_e[pl.ds(o, L)]
                prev = _take16(si, rot_prev)
                first = jnp.logical_or(iota == 0, si != prev)
                nxt_in = _take16(si, jnp.minimum(iota + 1, L - 1))
                last = jnp.logical_or(iota == L - 1, si != nxt_in)
                start = plsc.cummax(jnp.where(first, iota, 0))
                prev_ix = jnp.maximum(start - 1, 0)
                for c in range(CPT):
                    v = plsc.load_gather(xslice[c], [gi]) * wvv
                    cs = plsc.cumsum(v)
                    cprev = _take16(cs, prev_ix)
                    runsum = cs - jnp.where(start > 0, cprev,
                                            jnp.float32(0.0))
                    plsc.addupdate_scatter(oslice[c], [si], runsum,
                                           mask=last)
                return 0
            lax.fori_loop(jnp.int32(0), jnp.int32(GROUPS), blk, 0)

    def outer(i, _):
        for b in range(2):
            ci = i * 2 + jnp.int32(b)

            @pl.when(ci + 1 < jnp.int32(NCHUNKS))
            def _():
                start_chunk(ci + 1, 1 - b)

            wait_chunk(ci, b)
            process(ci, pv[b], wv_[b])
        return 0
    lax.fori_loop(jnp.int32(0), jnp.int32(NCHUNKS // 2), outer, 0)

    @plsc.parallel_loop(jnp.int32(0), jnp.int32(CPT * NODE_VECS),
                        jnp.int32(1), unroll=8,
                        carry=jnp.zeros((L,), jnp.float32))
    def acc(j, a):
        v = out_v[pl.ds(j * L, L)]
        return a + v * v
    acc_v[...] = acc
    pltpu.sync_copy(out_v, out_hbm.at[pl.ds(base_flat, CPT * N_NODES)])
    pltpu.sync_copy(acc_v, psum_hbm.at[pl.ds(wid * L, L)])


_sc_mesh = plsc.VectorSubcoreMesh(core_axis_name="c", subcore_axis_name="s")

_spmv = pl.kernel(
    _spmv_body,
    out_type=[
        jax.ShapeDtypeStruct((D_FEAT * N_NODES,), jnp.float32),
        jax.ShapeDtypeStruct((NW * L,), jnp.float32),
    ],
    mesh=_sc_mesh,
    compiler_params=pltpu.CompilerParams(needs_layout_passes=False),
    scratch_types=[
        pltpu.VMEM((CPT * N_NODES,), jnp.float32),
        pltpu.VMEM((CPT * N_NODES,), jnp.float32),
        pltpu.VMEM((CHUNK,), jnp.int32),
        pltpu.VMEM((CHUNK,), jnp.int32),
        pltpu.VMEM((CHUNK,), jnp.float32),
        pltpu.VMEM((CHUNK,), jnp.float32),
        pltpu.VMEM((NCHUNKS * L,), jnp.int32),
        pltpu.VMEM((L,), jnp.float32),
        pltpu.SemaphoreType.DMA,
        pltpu.SemaphoreType.DMA,
    ],
)


def _setup_body(b_ref, ax_ref, r_ref, bn2_ref, rn2_ref):
    bb = b_ref[...]
    rr = bb - ax_ref[...]
    r_ref[...] = rr
    bn2_ref[0, 0] = jnp.sum(bb * bb)
    rn2_ref[0, 0] = jnp.sum(rr * rr)


_tc_setup = pl.pallas_call(
    _setup_body,
    out_shape=[
        jax.ShapeDtypeStruct((D_FEAT, N_NODES), jnp.float32),
        jax.ShapeDtypeStruct((1, 1), jnp.float32),
        jax.ShapeDtypeStruct((1, 1), jnp.float32),
    ],
    out_specs=[
        pl.BlockSpec(memory_space=pltpu.VMEM),
        pl.BlockSpec(memory_space=pltpu.SMEM),
        pl.BlockSpec(memory_space=pltpu.SMEM),
    ],
)


def _update_body(a_ref, x_ref, p_ref, r_ref, q_ref, xo_ref, ro_ref, rn2_ref):
    a = a_ref[0, 0]
    xo_ref[...] = x_ref[...] + a * p_ref[...]
    rr = r_ref[...] - a * q_ref[...]
    ro_ref[...] = rr
    rn2_ref[0, 0] = jnp.sum(rr * rr)


_tc_update = pl.pallas_call(
    _update_body,
    in_specs=[
        pl.BlockSpec(memory_space=pltpu.SMEM),
        pl.BlockSpec(memory_space=pltpu.VMEM),
        pl.BlockSpec(memory_space=pltpu.VMEM),
        pl.BlockSpec(memory_space=pltpu.VMEM),
        pl.BlockSpec(memory_space=pltpu.VMEM),
    ],
    out_shape=[
        jax.ShapeDtypeStruct((D_FEAT, N_NODES), jnp.float32),
        jax.ShapeDtypeStruct((D_FEAT, N_NODES), jnp.float32),
        jax.ShapeDtypeStruct((1, 1), jnp.float32),
    ],
    out_specs=[
        pl.BlockSpec(memory_space=pltpu.VMEM),
        pl.BlockSpec(memory_space=pltpu.VMEM),
        pl.BlockSpec(memory_space=pltpu.SMEM),
    ],
)


def _pupd_body(c_ref, s_ref, p_ref, po_ref):
    po_ref[...] = c_ref[0, 0] * s_ref[...] + c_ref[0, 1] * p_ref[...]


_tc_pupd = pl.pallas_call(
    _pupd_body,
    in_specs=[
        pl.BlockSpec(memory_space=pltpu.SMEM),
        pl.BlockSpec(memory_space=pltpu.VMEM),
        pl.BlockSpec(memory_space=pltpu.VMEM),
    ],
    out_shape=jax.ShapeDtypeStruct((D_FEAT, N_NODES), jnp.float32),
    out_specs=pl.BlockSpec(memory_space=pltpu.VMEM),
)


def kernel(b, xref, edge_index, edge_weights):
    src = edge_index[0].astype(jnp.int32)
    dst = edge_index[1].astype(jnp.int32)
    w = edge_weights.astype(jnp.float32)
    E = w.shape[0]
    G = E // L

    def prep(scatter_idx, gather_idx):
        perm = jnp.argsort(scatter_idx)

        def stride(a):
            return a[perm].reshape(L, G).T.reshape(-1)

        si = stride(scatter_idx)
        sg = si.reshape(-1, L)
        gdup = jnp.any(sg[:, 1:] == sg[:, :-1], axis=1)
        cdup = jnp.any(gdup.reshape(NCHUNKS, GROUPS), axis=1).astype(jnp.int32)
        flags = jnp.repeat(cdup, L)
        packed = stride(gather_idx) | (si << 14)
        return packed, stride(w), flags

    pF, wF, fF = prep(dst, src)
    pA, wA, fA = prep(src, dst)

    bT = jnp.transpose(b)
    x = jnp.transpose(xref).reshape(-1)

    f32 = jnp.float32

    def spmv(vec_flat, pk, wi, fl):
        out, ps = _spmv(vec_flat, pk, wi, fl)
        return out, jnp.square(jnp.sqrt(jnp.sum(ps)))

    def as2d(flat):
        return flat.reshape(D_FEAT, N_NODES)

    ax, _ = spmv(x, pF, wF, fF)
    r2d, bn2, rn2 = _tc_setup(bT, as2d(ax))
    r = r2d.reshape(-1)
    bnorm = jnp.sqrt(bn2[0, 0])
    done = (jnp.sqrt(rn2[0, 0]) / bnorm) < EPS
    s, gamma = spmv(r, pA, wA, fA)
    p = s

    for _ in range(CGLS_IT):
        q, delta = spmv(p, pF, wF, fF)
        alpha = gamma / delta
        aeff = jnp.where(done, f32(0.0), alpha).astype(f32)
        x2d, r2d, rn2 = _tc_update(aeff.reshape(1, 1), as2d(x), as2d(p),
                                   as2d(r), as2d(q))
        x, r = x2d.reshape(-1), r2d.reshape(-1)
        done = jnp.logical_or(done, (jnp.sqrt(rn2[0, 0]) / bnorm) < EPS)
        s, gamma_new = spmv(r, pA, wA, fA)
        beta = gamma_new / gamma
        active = jnp.logical_not(done)
        c0 = jnp.where(active, f32(1.0), f32(0.0)).astype(f32)
        c1 = jnp.where(active, beta, f32(1.0)).astype(f32)
        p = _tc_pupd(jnp.stack([c0, c1]).reshape(1, 2), as2d(s),
                     as2d(p)).reshape(-1)
        gamma = jnp.where(active, gamma_new, gamma)

    return jnp.transpose(as2d(x)), jnp.transpose(as2d(r))


# speedup vs baseline: 6.0181x; 1.0477x over previous
import functools

import jax
import jax.numpy as jnp
from jax import lax
from jax.experimental import pallas as pl
from jax.experimental.pallas import tpu as pltpu
from jax.experimental.pallas import tpu_sc as plsc

N_NODES = 10000
D_FEAT = 128
N_EDGES = 320000
CGLS_IT = 10
EPS = 0.01

NC = 2
NS = 16
L = 16
NW = NC * NS
CPT = D_FEAT // NW
CHUNK = 6400
NCHUNKS = N_EDGES // CHUNK
GROUPS = CHUNK // L
NODE_VECS = N_NODES // L
UNROLL = 4


def _take16(x, idx):
    dnums = lax.GatherDimensionNumbers(
        offset_dims=(), collapsed_slice_dims=(0,), start_index_map=(0,))
    return lax.gather(x, idx[:, None], dnums, (1,),
                      mode=lax.GatherScatterMode.PROMISE_IN_BOUNDS)


def _spmv_body(x_hbm, pidx_hbm, w_hbm, flags_hbm, out_hbm,
               psum_hbm,
               x_v, out_v, p0_v, p1_v, w0_v, w1_v, f_v, acc_v,
               sem0, sem1):
    wid = lax.axis_index("s") * NC + lax.axis_index("c")
    base_flat = wid * (CPT * N_NODES)
    pv = (p0_v, p1_v)
    wv_ = (w0_v, w1_v)
    sems = (sem0, sem1)

    def start_chunk(ci, b):
        base = ci * CHUNK
        pltpu.async_copy(pidx_hbm.at[pl.ds(base, CHUNK)], pv[b], sems[b])
        pltpu.async_copy(w_hbm.at[pl.ds(base, CHUNK)], wv_[b], sems[b])

    def wait_chunk(ci, b):
        base = ci * CHUNK
        pltpu.make_async_copy(pidx_hbm.at[pl.ds(base, CHUNK)], pv[b],
                              sems[b]).wait()
        pltpu.make_async_copy(w_hbm.at[pl.ds(base, CHUNK)], wv_[b],
                              sems[b]).wait()

    start_chunk(jnp.int32(0), 0)

    pltpu.sync_copy(x_hbm.at[pl.ds(base_flat, CPT * N_NODES)], x_v)
    pltpu.sync_copy(flags_hbm, f_v)

    @plsc.parallel_loop(jnp.int32(0), jnp.int32(CPT * NODE_VECS),
                        jnp.int32(1), unroll=8)
    def _zero(j):
        out_v[pl.ds(j * L, L)] = jnp.zeros((L,), jnp.float32)

    iota = lax.iota(jnp.int32, L)
    lane_lt = iota < (L - 1)
    rot_prev = (iota + (L - 1)) % L
    rot_next = (iota + 1) % L
    xslice = [x_v.at[pl.ds(c * N_NODES, N_NODES)] for c in range(CPT)]
    oslice = [out_v.at[pl.ds(c * N_NODES, N_NODES)] for c in range(CPT)]

    def process(ci, p_e, w_e):
        fv = f_v[pl.ds(ci * L, L)]
        dup = jnp.max(fv) > 0

        @pl.when(jnp.logical_not(dup))
        def _():
            @plsc.parallel_loop(jnp.int32(0), jnp.int32(GROUPS),
                                jnp.int32(1), unroll=8)
            def _blk(j):
                o = j * L
                pk = p_e[pl.ds(o, L)]
                gi = pk & jnp.int32(0x3FFF)
                si = lax.shift_right_logical(pk, jnp.int32(14))
                wvv = w_e[pl.ds(o, L)]
                for c in range(CPT):
                    v = plsc.load_gather(xslice[c], [gi]) * wvv
                    plsc.addupdate_scatter(oslice[c], [si], v)

        @pl.when(dup)
        def _():
            def blk(j, _):
                o = j * L
                pk = p_e[pl.ds(o, L)]
                gi = pk & jnp.int32(0x3FFF)
                si = lax.shift_right_logical(pk, jnp.int32(14))
                wvv = w_e[pl.ds(o, L)]
                prev = _take16(si, rot_prev)
                first = jnp.logical_or(iota == 0, si != prev)
                nxt_in = _take16(si, jnp.minimum(iota + 1, L - 1))
                last = jnp.logical_or(iota == L - 1, si != nxt_in)
                start = plsc.cummax(jnp.where(first, iota, 0))
                prev_ix = jnp.maximum(start - 1, 0)
                for c in range(CPT):
                    v = plsc.load_gather(xslice[c], [gi]) * wvv
                    cs = plsc.cumsum(v)
                    cprev = _take16(cs, prev_ix)
                    runsum = cs - jnp.where(start > 0, cprev,
                                            jnp.float32(0.0))
                    plsc.addupdate_scatter(oslice[c], [si], runsum,
                                           mask=last)
                return 0
            lax.fori_loop(jnp.int32(0), jnp.int32(GROUPS), blk, 0)

    def outer(i, _):
        for b in range(2):
            ci = i * 2 + jnp.int32(b)

            @pl.when(ci + 1 < jnp.int32(NCHUNKS))
            def _():
                start_chunk(ci + 1, 1 - b)

            wait_chunk(ci, b)
            process(ci, pv[b], wv_[b])
        return 0
    lax.fori_loop(jnp.int32(0), jnp.int32(NCHUNKS // 2), outer, 0)

    @plsc.parallel_loop(jnp.int32(0), jnp.int32(CPT * NODE_VECS),
                        jnp.int32(1), unroll=8,
                        carry=jnp.zeros((L,), jnp.float32))
    def acc(j, a):
        v = out_v[pl.ds(j * L, L)]
        return a + v * v
    acc_v[...] = acc
    pltpu.sync_copy(out_v, out_hbm.at[pl.ds(base_flat, CPT * N_NODES)])
    pltpu.sync_copy(acc_v, psum_hbm.at[pl.ds(wid * L, L)])


_sc_mesh = plsc.VectorSubcoreMesh(core_axis_name="c", subcore_axis_name="s")

_spmv = pl.kernel(
    _spmv_body,
    out_type=[
        jax.ShapeDtypeStruct((D_FEAT * N_NODES,), jnp.float32),
        jax.ShapeDtypeStruct((NW * L,), jnp.float32),
    ],
    mesh=_sc_mesh,
    compiler_params=pltpu.CompilerParams(needs_layout_passes=False),
    scratch_types=[
        pltpu.VMEM((CPT * N_NODES,), jnp.float32),
        pltpu.VMEM((CPT * N_NODES,), jnp.float32),
        pltpu.VMEM((CHUNK,), jnp.int32),
        pltpu.VMEM((CHUNK,), jnp.int32),
        pltpu.VMEM((CHUNK,), jnp.float32),
        pltpu.VMEM((CHUNK,), jnp.float32),
        pltpu.VMEM((NCHUNKS * L,), jnp.int32),
        pltpu.VMEM((L,), jnp.float32),
        pltpu.SemaphoreType.DMA,
        pltpu.SemaphoreType.DMA,
    ],
)


def _setup_body(b_ref, ax_ref, r_ref, bn2_ref, rn2_ref):
    bb = b_ref[...]
    rr = bb - ax_ref[...]
    r_ref[...] = rr
    bn2_ref[0, 0] = jnp.sum(bb * bb)
    rn2_ref[0, 0] = jnp.sum(rr * rr)


_tc_setup = pl.pallas_call(
    _setup_body,
    out_shape=[
        jax.ShapeDtypeStruct((D_FEAT, N_NODES), jnp.float32),
        jax.ShapeDtypeStruct((1, 1), jnp.float32),
        jax.ShapeDtypeStruct((1, 1), jnp.float32),
    ],
    out_specs=[
        pl.BlockSpec(memory_space=pltpu.VMEM),
        pl.BlockSpec(memory_space=pltpu.SMEM),
        pl.BlockSpec(memory_space=pltpu.SMEM),
    ],
)


def _update_body(a_ref, x_ref, p_ref, r_ref, q_ref, xo_ref, ro_ref, rn2_ref):
    a = a_ref[0, 0]
    xo_ref[...] = x_ref[...] + a * p_ref[...]
    rr = r_ref[...] - a * q_ref[...]
    ro_ref[...] = rr
    rn2_ref[0, 0] = jnp.sum(rr * rr)


_tc_update = pl.pallas_call(
    _update_body,
    in_specs=[
        pl.BlockSpec(memory_space=pltpu.SMEM),
        pl.BlockSpec(memory_space=pltpu.VMEM),
        pl.BlockSpec(memory_space=pltpu.VMEM),
        pl.BlockSpec(memory_space=pltpu.VMEM),
        pl.BlockSpec(memory_space=pltpu.VMEM),
    ],
    out_shape=[
        jax.ShapeDtypeStruct((D_FEAT, N_NODES), jnp.float32),
        jax.ShapeDtypeStruct((D_FEAT, N_NODES), jnp.float32),
        jax.ShapeDtypeStruct((1, 1), jnp.float32),
    ],
    out_specs=[
        pl.BlockSpec(memory_space=pltpu.VMEM),
        pl.BlockSpec(memory_space=pltpu.VMEM),
        pl.BlockSpec(memory_space=pltpu.SMEM),
    ],
)


def _pupd_body(c_ref, s_ref, p_ref, po_ref):
    po_ref[...] = c_ref[0, 0] * s_ref[...] + c_ref[0, 1] * p_ref[...]


_tc_pupd = pl.pallas_call(
    _pupd_body,
    in_specs=[
        pl.BlockSpec(memory_space=pltpu.SMEM),
        pl.BlockSpec(memory_space=pltpu.VMEM),
        pl.BlockSpec(memory_space=pltpu.VMEM),
    ],
    out_shape=jax.ShapeDtypeStruct((D_FEAT, N_NODES), jnp.float32),
    out_specs=pl.BlockSpec(memory_space=pltpu.VMEM),
)


def kernel(b, xref, edge_index, edge_weights):
    src = edge_index[0].astype(jnp.int32)
    dst = edge_index[1].astype(jnp.int32)
    w = edge_weights.astype(jnp.float32)
    E = w.shape[0]
    G = E // L

    def prep(scatter_idx, gather_idx):
        ssc, sga, sw = lax.sort((scatter_idx, gather_idx, w), num_keys=1)
        packed_sorted = sga | (ssc << 14)

        def stride(a):
            return a.reshape(L, G).T.reshape(-1)

        packed = stride(packed_sorted)
        si = lax.shift_right_logical(packed, jnp.int32(14))
        sg = si.reshape(-1, L)
        gdup = jnp.any(sg[:, 1:] == sg[:, :-1], axis=1)
        cdup = jnp.any(gdup.reshape(NCHUNKS, GROUPS), axis=1).astype(jnp.int32)
        flags = jnp.repeat(cdup, L)
        return packed, stride(sw), flags

    pF, wF, fF = prep(dst, src)
    pA, wA, fA = prep(src, dst)

    bT = jnp.transpose(b)
    x = jnp.transpose(xref).reshape(-1)

    f32 = jnp.float32

    def spmv(vec_flat, pk, wi, fl):
        out, ps = _spmv(vec_flat, pk, wi, fl)
        return out, jnp.square(jnp.sqrt(jnp.sum(ps)))

    def as2d(flat):
        return flat.reshape(D_FEAT, N_NODES)

    ax, _ = spmv(x, pF, wF, fF)
    r2d, bn2, rn2 = _tc_setup(bT, as2d(ax))
    r = r2d.reshape(-1)
    bnorm = jnp.sqrt(bn2[0, 0])
    done = (jnp.sqrt(rn2[0, 0]) / bnorm) < EPS
    s, gamma = spmv(r, pA, wA, fA)
    p = s

    for _ in range(CGLS_IT):
        q, delta = spmv(p, pF, wF, fF)
        alpha = gamma / delta
        aeff = jnp.where(done, f32(0.0), alpha).astype(f32)
        x2d, r2d, rn2 = _tc_update(aeff.reshape(1, 1), as2d(x), as2d(p),
                                   as2d(r), as2d(q))
        x, r = x2d.reshape(-1), r2d.reshape(-1)
        done = jnp.logical_or(done, (jnp.sqrt(rn2[0, 0]) / bnorm) < EPS)
        s, gamma_new = spmv(r, pA, wA, fA)
        beta = gamma_new / gamma
        active = jnp.logical_not(done)
        c0 = jnp.where(active, f32(1.0), f32(0.0)).astype(f32)
        c1 = jnp.where(active, beta, f32(1.0)).astype(f32)
        p = _tc_pupd(jnp.stack([c0, c1]).reshape(1, 2), as2d(s),
                     as2d(p)).reshape(-1)
        gamma = jnp.where(active, gamma_new, gamma)

    return jnp.transpose(as2d(x)), jnp.transpose(as2d(r))
